# Initial kernel scaffold; baseline (speedup 1.0000x reference)
import jax
import jax.numpy as jnp
from jax.experimental import pallas as pl


def kernel(x_all, edge_index_all, weight, att, bias):
    raise NotImplementedError("write your pallas kernel here")


# speedup vs baseline: 10.3394x; 10.3394x over previous
import functools
import math

import jax
import jax.numpy as jnp
from jax import lax
from jax.experimental import pallas as pl
from jax.experimental.pallas import tpu as pltpu
from jax.experimental.pallas import tpu_sc as plsc

T = 3
N = 10000
E = 320000
D = 128
OUT = 128
NEG = 0.2
NBSZ = 15
AGG = 0.8

EP = E + N
CHUNK = 256
NW = 32
EPAD = ((EP + NW * CHUNK - 1) // (NW * CHUNK)) * (NW * CHUNK)
NROW = EPAD // 128
NPAD = 10240

_Q = 1.0 / (1.0 + math.exp(-1.0 / NBSZ))
_LOG_Q = math.log(_Q)
_LOG_1MQ = math.log1p(-_Q)
_W1 = 1.0 / (1.0 + math.exp(-1.0))


def _prep_body(x_ref, w_ref, attm_ref, xw_ref, s_ref):
    xw = jnp.dot(x_ref[...], w_ref[...], preferred_element_type=jnp.float32)
    xw_ref[...] = xw
    s_ref[...] = jnp.dot(xw, attm_ref[...], preferred_element_type=jnp.float32)


def _prep(x2d, weight, attm):
    B = 400
    g = (T * N) // B
    return pl.pallas_call(
        _prep_body,
        grid=(g,),
        in_specs=[
            pl.BlockSpec((B, D), lambda i: (i, 0)),
            pl.BlockSpec((D, OUT), lambda i: (0, 0)),
            pl.BlockSpec((OUT, 2), lambda i: (0, 0)),
        ],
        out_specs=[
            pl.BlockSpec((B, OUT), lambda i: (i, 0)),
            pl.BlockSpec((B, 2), lambda i: (i, 0)),
        ],
        out_shape=[
            jax.ShapeDtypeStruct((T * N, OUT), jnp.float32),
            jax.ShapeDtypeStruct((T * N, 2), jnp.float32),
        ],
    )(x2d, weight, attm)


def _edge_core_jnp(src, dst, valid, xw, s_dst, s_src):
    deg = jax.ops.segment_sum(valid, dst, num_segments=N)
    alpha = s_dst[dst] + s_src[src]
    alpha = jnp.maximum(alpha, NEG * alpha)
    p = jnp.clip(1.0 / (1.0 + jnp.exp(-alpha)), 0.01, 0.99)
    a = p * valid / (deg[dst] + 1e-16)
    out = jax.ops.segment_sum(a[:, None] * xw[src], dst, num_segments=N)
    return out, p


RPT = NROW // 16
RPW = NROW // NW
NSL = NPAD // 16
ROWS_T = N // 16


def _att_body(enc_hbm, sd_hbm, ss_hbm, p_hbm, a_hbm,
              sd_v, ss_v, inv_v, deg_v, comb_v, invsl_v, encb, pb, ab,
              spm_deg, spm_inv):
    cid = lax.axis_index("c")
    sid = lax.axis_index("s")
    wid = sid * 2 + cid
    zed = jnp.zeros((16,), jnp.float32)

    pltpu.sync_copy(sd_hbm, sd_v)
    pltpu.sync_copy(ss_hbm, ss_v)

    def zero_deg(i, _):
        deg_v[pl.ds(i * 16, 16)] = zed
        return 0
    lax.fori_loop(0, NPAD // 16, zero_deg, 0)

    def dega(ch, _):
        row0 = sid * RPT + ch * 2
        pltpu.sync_copy(enc_hbm.at[pl.ds(row0, 2)], encb)
        for j in range(2):
            for k in range(8):
                enc = encb[j, pl.ds(k * 16, 16)]
                d16 = lax.shift_right_logical(enc, 1) & 16383
                v16 = (enc & 1).astype(jnp.float32)
                plsc.addupdate_scatter(deg_v, [d16], v16)
        return 0
    lax.fori_loop(0, RPT // 2, dega, 0)

    pltpu.sync_copy(deg_v, spm_deg.at[sid])
    plsc.subcore_barrier()

    pltpu.sync_copy(spm_deg.at[:, pl.ds(sid * NSL, NSL)], comb_v)

    def comb(jj, _):
        acc = comb_v[0, pl.ds(jj * 16, 16)]
        for i in range(1, 16):
            acc = acc + comb_v[i, pl.ds(jj * 16, 16)]
        invsl_v[pl.ds(jj * 16, 16)] = 1.0 / (acc + 1e-16)
        return 0
    lax.fori_loop(0, NSL // 16, comb, 0)
    pltpu.sync_copy(invsl_v, spm_inv.at[pl.ds(sid * NSL, NSL)])
    plsc.subcore_barrier()
    pltpu.sync_copy(spm_inv, inv_v)

    def attn(ch, _):
        row0 = wid * RPW + ch * 2
        pltpu.sync_copy(enc_hbm.at[pl.ds(row0, 2)], encb)
        for j in range(2):
            for k in range(8):
                enc = encb[j, pl.ds(k * 16, 16)]
                s16 = lax.shift_right_logical(enc, 15)
                d16 = lax.shift_right_logical(enc, 1) & 16383
                v16 = (enc & 1).astype(jnp.float32)
                alpha = plsc.load_gather(sd_v, [d16]) + plsc.load_gather(ss_v, [s16])
                alpha = jnp.maximum(alpha, NEG * alpha)
                pv = 1.0 / (1.0 + jnp.exp(-alpha))
                pv = jnp.minimum(jnp.maximum(pv, 0.01), 0.99)
                pb[j, pl.ds(k * 16, 16)] = pv
                ab[j, pl.ds(k * 16, 16)] = pv * v16 * plsc.load_gather(inv_v, [d16])
        pltpu.sync_copy(pb, p_hbm.at[pl.ds(row0, 2)])
        pltpu.sync_copy(ab, a_hbm.at[pl.ds(row0, 2)])
        return 0
    lax.fori_loop(0, RPW // 2, attn, 0)


def _msg_body(enc_hbm, xw_hbm, a_hbm, parts_hbm,
              encb, srcb, dstb, af, rows, spm_out, sem):
    cid = lax.axis_index("c")
    sid = lax.axis_index("s")
    wid = sid * 2 + cid
    zed = jnp.zeros((16,), jnp.float32)

    def zrows(i, _):
        for k in range(8):
            rows[i, pl.ds(k * 16, 16)] = zed
        return 0
    lax.fori_loop(0, CHUNK, zrows, 0)
    pltpu.sync_copy(rows, spm_out.at[pl.ds(sid * ROWS_T, CHUNK)])
    pltpu.sync_copy(rows, spm_out.at[pl.ds(sid * ROWS_T + CHUNK, CHUNK)])
    pltpu.sync_copy(rows.at[pl.ds(0, ROWS_T - 2 * CHUNK)],
                    spm_out.at[pl.ds(sid * ROWS_T + 2 * CHUNK, ROWS_T - 2 * CHUNK)])
    plsc.subcore_barrier()

    def chunk_body(ch, _):
        row0 = wid * RPW + ch * 2
        pltpu.sync_copy(enc_hbm.at[pl.ds(row0, 2)], encb)
        for j in range(2):
            pltpu.sync_copy(a_hbm.at[row0 + j], af.at[pl.ds(j * 128, 128)])
        for j in range(2):
            for k in range(8):
                enc = encb[j, pl.ds(k * 16, 16)]
                srcb[j, pl.ds(k * 16, 16)] = lax.shift_right_logical(enc, 15)
                dstb[j, pl.ds(k * 16, 16)] = lax.shift_right_logical(enc, 1) & 16383
        for j in range(2):
            pltpu.async_copy(xw_hbm.at[srcb.at[j]],
                             rows.at[pl.ds(j * 128, 128)], sem).wait()

        def scale(e, _):
            spl = plsc.load_gather(af, [jnp.full((16,), 0, jnp.int32) + e])
            for k in range(8):
                rows[e, pl.ds(k * 16, 16)] = rows[e, pl.ds(k * 16, 16)] * spl
            return 0
        lax.fori_loop(0, CHUNK, scale, 0)
        for j in range(2):
            pltpu.sync_copy(rows.at[pl.ds(j * 128, 128)],
                            spm_out.at[dstb.at[j]], add=True)
        return 0
    lax.fori_loop(0, RPW // 2, chunk_body, 0)
    plsc.subcore_barrier()

    for off, sz in ((0, CHUNK), (CHUNK, CHUNK), (2 * CHUNK, ROWS_T - 2 * CHUNK)):
        pltpu.sync_copy(spm_out.at[pl.ds(sid * ROWS_T + off, sz)],
                        rows.at[pl.ds(0, sz)])
        pltpu.sync_copy(rows.at[pl.ds(0, sz)],
                        parts_hbm.at[cid, pl.ds(sid * ROWS_T + off, sz)])


_SC_PARAMS = pltpu.CompilerParams(use_tc_tiling_on_sc=False,
                                  needs_layout_passes=False)


def _edge_core_sc(enc2d, xw, sd_pad, ss_pad):
    mesh = plsc.VectorSubcoreMesh(core_axis_name="c", subcore_axis_name="s")
    p2d, a2d = pl.kernel(
        _att_body,
        out_type=[
            jax.ShapeDtypeStruct((NROW, 128), jnp.float32),
            jax.ShapeDtypeStruct((NROW, 128), jnp.float32),
        ],
        mesh=mesh,
        scratch_types=[
            pltpu.VMEM((NPAD,), jnp.float32),
            pltpu.VMEM((NPAD,), jnp.float32),
            pltpu.VMEM((NPAD,), jnp.float32),
            pltpu.VMEM((NPAD,), jnp.float32),
            pltpu.VMEM((16, NSL), jnp.float32),
            pltpu.VMEM((NSL,), jnp.float32),
            pltpu.VMEM((2, 128), jnp.int32),
            pltpu.VMEM((2, 128), jnp.float32),
            pltpu.VMEM((2, 128), jnp.float32),
            pltpu.VMEM_SHARED((16, NPAD), jnp.float32),
            pltpu.VMEM_SHARED((NPAD,), jnp.float32),
        ],
        compiler_params=_SC_PARAMS,
    )(enc2d, sd_pad, ss_pad)
    parts = pl.kernel(
        _msg_body,
        out_type=jax.ShapeDtypeStruct((2, N, 128), jnp.float32),
        mesh=mesh,
        scratch_types=[
            pltpu.VMEM((2, 128), jnp.int32),
            pltpu.VMEM((2, 128), jnp.int32),
            pltpu.VMEM((2, 128), jnp.int32),
            pltpu.VMEM((CHUNK,), jnp.float32),
            pltpu.VMEM((CHUNK, 128), jnp.float32),
            pltpu.VMEM_SHARED((N, 128), jnp.float32),
            pltpu.SemaphoreType.DMA,
        ],
        compiler_params=_SC_PARAMS,
    )(enc2d, xw, a2d)
    return parts, p2d


def _kl_body(p_ref, v_ref, kl_ref, vs_ref):
    j = pl.program_id(1)

    @pl.when(j == 0)
    def _():
        kl_ref[...] = jnp.zeros_like(kl_ref)
        vs_ref[...] = jnp.zeros_like(vs_ref)

    p = p_ref[...]
    v = v_ref[...]
    kl = p * (jnp.log(p) - _LOG_Q) + (1.0 - p) * (jnp.log1p(-p) - _LOG_1MQ)
    kl_ref[...] += jnp.sum(kl * v).reshape(1, 1, 1)
    vs_ref[...] += jnp.sum(v).reshape(1, 1, 1)


def _kl_sums(p3d, v3d):
    BR = 64
    g = NROW // BR
    return pl.pallas_call(
        _kl_body,
        grid=(T, g),
        in_specs=[
            pl.BlockSpec((1, BR, 128), lambda t, j: (t, j, 0)),
            pl.BlockSpec((1, BR, 128), lambda t, j: (t, j, 0)),
        ],
        out_specs=[
            pl.BlockSpec((1, 1, 1), lambda t, j: (t, 0, 0)),
            pl.BlockSpec((1, 1, 1), lambda t, j: (t, 0, 0)),
        ],
        out_shape=[
            jax.ShapeDtypeStruct((T, 1, 1), jnp.float32),
            jax.ShapeDtypeStruct((T, 1, 1), jnp.float32),
        ],
    )(p3d, v3d)


def _mix_body(parts_ref, bias_ref, out_ref):
    x = parts_ref[...]
    b = bias_ref[...]
    raw0 = x[0, 0] + x[0, 1] + b
    raw1 = x[1, 0] + x[1, 1] + b
    raw2 = x[2, 0] + x[2, 1] + b
    o0 = raw0
    o1 = 0.5 * AGG * o0 + (1.0 - AGG) * raw1
    o2 = 0.5 * AGG / 2.0 * o0 + _W1 * AGG / 2.0 * o1 + (1.0 - AGG) * raw2
    out_ref[...] = jnp.stack([o0, o1, o2], axis=0)


def _mix(parts, bias2d):
    B = 400
    g = N // B
    return pl.pallas_call(
        _mix_body,
        grid=(g,),
        in_specs=[
            pl.BlockSpec((T, 2, B, 128), lambda i: (0, 0, i, 0)),
            pl.BlockSpec((1, 128), lambda i: (0, 0)),
        ],
        out_specs=pl.BlockSpec((T, B, 128), lambda i: (0, i, 0)),
        out_shape=jax.ShapeDtypeStruct((T, N, 128), jnp.float32),
    )(parts, bias2d)


def kernel(x_all, edge_index_all, weight, att, bias):
    x2d = x_all.reshape(T * N, D)
    attm = jnp.stack([att[0, 0, :OUT], att[0, 0, OUT:]], axis=1)
    xw2d, s2d = _prep(x2d, weight, attm)
    xw_all = xw2d.reshape(T, N, OUT)
    s_dst_all = s2d[:, 0].reshape(T, N)
    s_src_all = s2d[:, 1].reshape(T, N)

    loop = jnp.arange(N, dtype=jnp.int32)
    pad = jnp.zeros((EPAD - EP,), dtype=jnp.int32)
    parts = []
    ps = []
    vs = []
    for t in range(T):
        e = edge_index_all[t]
        src = jnp.concatenate([e[0], loop, pad])
        dst = jnp.concatenate([e[1], loop, pad])
        valid = jnp.concatenate([
            (e[0] != e[1]).astype(jnp.float32),
            jnp.ones((N,), jnp.float32),
            jnp.zeros((EPAD - EP,), jnp.float32),
        ])
        sd_pad = jnp.pad(s_dst_all[t], (0, NPAD - N))
        ss_pad = jnp.pad(s_src_all[t], (0, NPAD - N))
        enc = src * 32768 + dst * 2 + valid.astype(jnp.int32)
        parts_t, p2d = _edge_core_sc(enc.reshape(NROW, 128),
                                     xw_all[t], sd_pad, ss_pad)
        parts.append(parts_t)
        ps.append(p2d)
        vs.append(valid.reshape(NROW, 128))

    parts = jnp.stack(parts, axis=0)
    p3d = jnp.stack(ps, axis=0)
    v3d = jnp.stack(vs, axis=0)

    kl_sum, v_sum = _kl_sums(p3d, v3d)
    skl_mean = jnp.mean(kl_sum[:, 0, 0] / v_sum[:, 0, 0])
    out_list = _mix(parts, bias.reshape(1, 128))
    zero = jnp.zeros((), jnp.float32)
    return (out_list, zero, skl_mean, zero)


# speedup vs baseline: 10.7468x; 1.0394x over previous
import functools
import math

import jax
import jax.numpy as jnp
from jax import lax
from jax.experimental import pallas as pl
from jax.experimental.pallas import tpu as pltpu
from jax.experimental.pallas import tpu_sc as plsc

T = 3
N = 10000
E = 320000
D = 128
OUT = 128
NEG = 0.2
NBSZ = 15
AGG = 0.8

EP = E + N
CHUNK = 256
NW = 32
EPAD = ((EP + NW * CHUNK - 1) // (NW * CHUNK)) * (NW * CHUNK)
NROW = EPAD // 128
NPAD = 10240
ROWS_ALL = 10368

_Q = 1.0 / (1.0 + math.exp(-1.0 / NBSZ))
_LOG_Q = math.log(_Q)
_LOG_1MQ = math.log1p(-_Q)
_W1 = 1.0 / (1.0 + math.exp(-1.0))


def _prep_body(x_ref, w_ref, attm_ref, xw_ref, s_ref):
    xw = jnp.dot(x_ref[...], w_ref[...], preferred_element_type=jnp.float32)
    xw_ref[...] = xw
    s_ref[...] = jnp.dot(xw, attm_ref[...], preferred_element_type=jnp.float32)


def _prep(x2d, weight, attm):
    B = 400
    g = (T * N) // B
    return pl.pallas_call(
        _prep_body,
        grid=(g,),
        in_specs=[
            pl.BlockSpec((B, D), lambda i: (i, 0)),
            pl.BlockSpec((D, OUT), lambda i: (0, 0)),
            pl.BlockSpec((OUT, 2), lambda i: (0, 0)),
        ],
        out_specs=[
            pl.BlockSpec((B, OUT), lambda i: (i, 0)),
            pl.BlockSpec((B, 2), lambda i: (i, 0)),
        ],
        out_shape=[
            jax.ShapeDtypeStruct((T * N, OUT), jnp.float32),
            jax.ShapeDtypeStruct((T * N, 2), jnp.float32),
        ],
    )(x2d, weight, attm)


def _edge_core_jnp(src, dst, valid, xw, s_dst, s_src):
    deg = jax.ops.segment_sum(valid, dst, num_segments=N)
    alpha = s_dst[dst] + s_src[src]
    alpha = jnp.maximum(alpha, NEG * alpha)
    p = jnp.clip(1.0 / (1.0 + jnp.exp(-alpha)), 0.01, 0.99)
    a = p * valid / (deg[dst] + 1e-16)
    out = jax.ops.segment_sum(a[:, None] * xw[src], dst, num_segments=N)
    return out, p


RPT = NROW // 16
RPW = NROW // NW
NSL = NPAD // 16
ROWS_T = N // 16


def _att_body(enc_hbm, sd_hbm, ss_hbm, p_hbm, a_hbm,
              sd_v, ss_v, inv_v, deg_v, comb_v, invsl_v, encb, pb, ab,
              spm_deg, spm_inv):
    cid = lax.axis_index("c")
    sid = lax.axis_index("s")
    wid = sid * 2 + cid
    zed = jnp.zeros((16,), jnp.float32)

    pltpu.sync_copy(sd_hbm, sd_v)
    pltpu.sync_copy(ss_hbm, ss_v)

    def zero_deg(i, _):
        deg_v[pl.ds(i * 16, 16)] = zed
        return 0
    lax.fori_loop(0, NPAD // 16, zero_deg, 0)

    def dega(ch, _):
        row0 = sid * RPT + ch * 2
        pltpu.sync_copy(enc_hbm.at[pl.ds(row0, 2)], encb)
        for j in range(2):
            for k in range(8):
                enc = encb[j, pl.ds(k * 16, 16)]
                d16 = lax.shift_right_logical(enc, 1) & 16383
                v16 = (enc & 1).astype(jnp.float32)
                plsc.addupdate_scatter(deg_v, [d16], v16)
        return 0
    lax.fori_loop(0, RPT // 2, dega, 0)

    pltpu.sync_copy(deg_v, spm_deg.at[sid])
    plsc.subcore_barrier()

    pltpu.sync_copy(spm_deg.at[:, pl.ds(sid * NSL, NSL)], comb_v)

    def comb(jj, _):
        acc = comb_v[0, pl.ds(jj * 16, 16)]
        for i in range(1, 16):
            acc = acc + comb_v[i, pl.ds(jj * 16, 16)]
        invsl_v[pl.ds(jj * 16, 16)] = 1.0 / (acc + 1e-16)
        return 0
    lax.fori_loop(0, NSL // 16, comb, 0)
    pltpu.sync_copy(invsl_v, spm_inv.at[pl.ds(sid * NSL, NSL)])
    plsc.subcore_barrier()
    pltpu.sync_copy(spm_inv, inv_v)

    def attn(ch, _):
        row0 = wid * RPW + ch * 2
        pltpu.sync_copy(enc_hbm.at[pl.ds(row0, 2)], encb)
        for j in range(2):
            for k in range(8):
                enc = encb[j, pl.ds(k * 16, 16)]
                s16 = lax.shift_right_logical(enc, 15)
                d16 = lax.shift_right_logical(enc, 1) & 16383
                v16 = (enc & 1).astype(jnp.float32)
                alpha = plsc.load_gather(sd_v, [d16]) + plsc.load_gather(ss_v, [s16])
                alpha = jnp.maximum(alpha, NEG * alpha)
                pv = 1.0 / (1.0 + jnp.exp(-alpha))
                pv = jnp.minimum(jnp.maximum(pv, 0.01), 0.99)
                pb[j, pl.ds(k * 16, 16)] = pv
                ab[j, pl.ds(k * 16, 16)] = pv * v16 * plsc.load_gather(inv_v, [d16])
        pltpu.sync_copy(pb, p_hbm.at[pl.ds(row0, 2)])
        pltpu.sync_copy(ab, a_hbm.at[pl.ds(row0, 2)])
        return 0
    lax.fori_loop(0, RPW // 2, attn, 0)


def _msg_body(enc_hbm, xw_hbm, a_hbm, parts_hbm,
              encb, srcb, dstb, af0, rows0,
              spm_out, gsem0):
    cid = lax.axis_index("c")
    sid = lax.axis_index("s")
    wid = sid * 2 + cid
    zed = jnp.zeros((16,), jnp.float32)

    def zrows(i, _):
        for k in range(8):
            rows0[i, pl.ds(k * 16, 16)] = zed
        return 0
    lax.fori_loop(0, CHUNK, zrows, 0)
    pltpu.sync_copy(rows0, spm_out.at[pl.ds(sid * ROWS_T, CHUNK)])
    pltpu.sync_copy(rows0, spm_out.at[pl.ds(sid * ROWS_T + CHUNK, CHUNK)])
    pltpu.sync_copy(rows0.at[pl.ds(0, ROWS_T - 2 * CHUNK)],
                    spm_out.at[pl.ds(sid * ROWS_T + 2 * CHUNK, ROWS_T - 2 * CHUNK)])
    plsc.subcore_barrier()

    def chunk_body(ch, _):
        row0 = wid * RPW + ch * 2
        pltpu.sync_copy(enc_hbm.at[pl.ds(row0, 2)], encb)
        for j in range(2):
            pltpu.sync_copy(a_hbm.at[row0 + j], af0.at[pl.ds(j * 128, 128)])
        for j in range(2):
            for k in range(8):
                enc = encb[j, pl.ds(k * 16, 16)]
                srcb[0, j, pl.ds(k * 16, 16)] = lax.shift_right_logical(enc, 15)
                dstb[0, j, pl.ds(k * 16, 16)] = lax.shift_right_logical(enc, 1) & 16383
        d0 = pltpu.async_copy(xw_hbm.at[srcb.at[0, 0]],
                              rows0.at[pl.ds(0, 128)], gsem0)
        d1 = pltpu.async_copy(xw_hbm.at[srcb.at[0, 1]],
                              rows0.at[pl.ds(128, 128)], gsem0)
        d0.wait()
        d1.wait()

        def scale(e4, _):
            for u in range(4):
                e = e4 * 4 + u
                spl = plsc.load_gather(af0, [jnp.full((16,), 0, jnp.int32) + e])
                for k in range(8):
                    rows0[e, pl.ds(k * 16, 16)] = rows0[e, pl.ds(k * 16, 16)] * spl
            return 0
        lax.fori_loop(0, CHUNK // 4, scale, 0)
        for j in range(2):
            pltpu.sync_copy(rows0.at[pl.ds(j * 128, 128)],
                            spm_out.at[dstb.at[0, j]], add=True)
        return 0
    lax.fori_loop(0, RPW // 2, chunk_body, 0)
    plsc.subcore_barrier()

    for off, sz in ((0, CHUNK), (CHUNK, CHUNK), (2 * CHUNK, ROWS_T - 2 * CHUNK)):
        pltpu.sync_copy(spm_out.at[pl.ds(sid * ROWS_T + off, sz)],
                        rows0.at[pl.ds(0, sz)])
        pltpu.sync_copy(rows0.at[pl.ds(0, sz)],
                        parts_hbm.at[cid, pl.ds(sid * ROWS_T + off, sz)])


_SC_PARAMS = pltpu.CompilerParams(use_tc_tiling_on_sc=False,
                                  needs_layout_passes=False)


def _edge_core_sc(enc2d, xw, sd_pad, ss_pad):
    mesh = plsc.VectorSubcoreMesh(core_axis_name="c", subcore_axis_name="s")
    p2d, a2d = pl.kernel(
        _att_body,
        out_type=[
            jax.ShapeDtypeStruct((NROW, 128), jnp.float32),
            jax.ShapeDtypeStruct((NROW, 128), jnp.float32),
        ],
        mesh=mesh,
        scratch_types=[
            pltpu.VMEM((NPAD,), jnp.float32),
            pltpu.VMEM((NPAD,), jnp.float32),
            pltpu.VMEM((NPAD,), jnp.float32),
            pltpu.VMEM((NPAD,), jnp.float32),
            pltpu.VMEM((16, NSL), jnp.float32),
            pltpu.VMEM((NSL,), jnp.float32),
            pltpu.VMEM((2, 128), jnp.int32),
            pltpu.VMEM((2, 128), jnp.float32),
            pltpu.VMEM((2, 128), jnp.float32),
            pltpu.VMEM_SHARED((16, NPAD), jnp.float32),
            pltpu.VMEM_SHARED((NPAD,), jnp.float32),
        ],
        compiler_params=_SC_PARAMS,
    )(enc2d, sd_pad, ss_pad)
    parts = pl.kernel(
        _msg_body,
        out_type=jax.ShapeDtypeStruct((2, N, 128), jnp.float32),
        mesh=mesh,
        scratch_types=[
            pltpu.VMEM((2, 128), jnp.int32),
            pltpu.VMEM((2, 2, 128), jnp.int32),
            pltpu.VMEM((2, 2, 128), jnp.int32),
            pltpu.VMEM((CHUNK,), jnp.float32),
            pltpu.VMEM((CHUNK, 128), jnp.float32),
            pltpu.VMEM_SHARED((N, 128), jnp.float32),
            pltpu.SemaphoreType.DMA,
        ],
        compiler_params=_SC_PARAMS,
    )(enc2d, xw, a2d)
    return parts, p2d


def _kl_body(p_ref, v_ref, kl_ref, vs_ref):
    j = pl.program_id(1)

    @pl.when(j == 0)
    def _():
        kl_ref[...] = jnp.zeros_like(kl_ref)
        vs_ref[...] = jnp.zeros_like(vs_ref)

    p = p_ref[...]
    v = v_ref[...]
    kl = p * (jnp.log(p) - _LOG_Q) + (1.0 - p) * (jnp.log1p(-p) - _LOG_1MQ)
    kl_ref[...] += jnp.sum(kl * v).reshape(1, 1, 1)
    vs_ref[...] += jnp.sum(v).reshape(1, 1, 1)


def _kl_sums(p3d, v3d):
    BR = 64
    g = NROW // BR
    return pl.pallas_call(
        _kl_body,
        grid=(T, g),
        in_specs=[
            pl.BlockSpec((1, BR, 128), lambda t, j: (t, j, 0)),
            pl.BlockSpec((1, BR, 128), lambda t, j: (t, j, 0)),
        ],
        out_specs=[
            pl.BlockSpec((1, 1, 1), lambda t, j: (t, 0, 0)),
            pl.BlockSpec((1, 1, 1), lambda t, j: (t, 0, 0)),
        ],
        out_shape=[
            jax.ShapeDtypeStruct((T, 1, 1), jnp.float32),
            jax.ShapeDtypeStruct((T, 1, 1), jnp.float32),
        ],
    )(p3d, v3d)


def _mix_body(parts_ref, bias_ref, out_ref):
    x = parts_ref[...]
    b = bias_ref[...]
    raw0 = x[0, 0] + x[0, 1] + b
    raw1 = x[1, 0] + x[1, 1] + b
    raw2 = x[2, 0] + x[2, 1] + b
    o0 = raw0
    o1 = 0.5 * AGG * o0 + (1.0 - AGG) * raw1
    o2 = 0.5 * AGG / 2.0 * o0 + _W1 * AGG / 2.0 * o1 + (1.0 - AGG) * raw2
    out_ref[...] = jnp.stack([o0, o1, o2], axis=0)


def _mix(parts, bias2d):
    B = 400
    g = N // B
    return pl.pallas_call(
        _mix_body,
        grid=(g,),
        in_specs=[
            pl.BlockSpec((T, 2, B, 128), lambda i: (0, 0, i, 0)),
            pl.BlockSpec((1, 128), lambda i: (0, 0)),
        ],
        out_specs=pl.BlockSpec((T, B, 128), lambda i: (0, i, 0)),
        out_shape=jax.ShapeDtypeStruct((T, N, 128), jnp.float32),
    )(parts, bias2d)


def kernel(x_all, edge_index_all, weight, att, bias):
    x2d = x_all.reshape(T * N, D)
    attm = jnp.stack([att[0, 0, :OUT], att[0, 0, OUT:]], axis=1)
    xw2d, s2d = _prep(x2d, weight, attm)
    xw_all = xw2d.reshape(T, N, OUT)
    s_dst_all = s2d[:, 0].reshape(T, N)
    s_src_all = s2d[:, 1].reshape(T, N)

    loop = jnp.arange(N, dtype=jnp.int32)
    pad = jnp.zeros((EPAD - EP,), dtype=jnp.int32)
    parts = []
    ps = []
    vs = []
    encs = []
    valids = []
    for t in range(T):
        e = edge_index_all[t]
        src = jnp.concatenate([e[0], loop, pad])
        dst = jnp.concatenate([e[1], loop, pad])
        valid = jnp.concatenate([
            (e[0] != e[1]).astype(jnp.float32),
            jnp.ones((N,), jnp.float32),
            jnp.zeros((EPAD - EP,), jnp.float32),
        ])
        encs.append((src * 32768 + dst * 2 + valid.astype(jnp.int32)).reshape(NROW, 128))
        valids.append(valid.reshape(NROW, 128))
    for t in range(T):
        sd_pad = jnp.pad(s_dst_all[t], (0, NPAD - N))
        ss_pad = jnp.pad(s_src_all[t], (0, NPAD - N))
        parts_t, p2d = _edge_core_sc(encs[t], xw_all[t], sd_pad, ss_pad)
        parts.append(parts_t)
        ps.append(p2d)

    parts = jnp.stack(parts, axis=0)
    p3d = jnp.stack(ps, axis=0)
    v3d = jnp.stack(valids, axis=0)

    kl_sum, v_sum = _kl_sums(p3d, v3d)
    skl_mean = jnp.mean(kl_sum[:, 0, 0] / v_sum[:, 0, 0])
    out_list = _mix(parts, bias.reshape(1, 128))
    zero = jnp.zeros((), jnp.float32)
    return (out_list, zero, skl_mean, zero)


# speedup vs baseline: 11.2312x; 1.0451x over previous
import functools
import math

import jax
import jax.numpy as jnp
from jax import lax
from jax.experimental import pallas as pl
from jax.experimental.pallas import tpu as pltpu
from jax.experimental.pallas import tpu_sc as plsc

T = 3
N = 10000
E = 320000
D = 128
OUT = 128
NEG = 0.2
NBSZ = 15
AGG = 0.8

EP = E + N
CHUNK = 256
NW = 32
EPAD = ((EP + NW * CHUNK - 1) // (NW * CHUNK)) * (NW * CHUNK)
NROW = EPAD // 128
NPAD = 10240
ROWS_ALL = 10368

_Q = 1.0 / (1.0 + math.exp(-1.0 / NBSZ))
_LOG_Q = math.log(_Q)
_LOG_1MQ = math.log1p(-_Q)
_W1 = 1.0 / (1.0 + math.exp(-1.0))


def _prep_body(x_ref, w_ref, attm_ref, xw_ref, s_ref):
    xw = jnp.dot(x_ref[...], w_ref[...], preferred_element_type=jnp.float32)
    xw_ref[...] = xw
    s_ref[...] = jnp.dot(xw, attm_ref[...], preferred_element_type=jnp.float32)


def _prep(x2d, weight, attm):
    B = 400
    g = (T * N) // B
    return pl.pallas_call(
        _prep_body,
        grid=(g,),
        in_specs=[
            pl.BlockSpec((B, D), lambda i: (i, 0)),
            pl.BlockSpec((D, OUT), lambda i: (0, 0)),
            pl.BlockSpec((OUT, 2), lambda i: (0, 0)),
        ],
        out_specs=[
            pl.BlockSpec((B, OUT), lambda i: (i, 0)),
            pl.BlockSpec((B, 2), lambda i: (i, 0)),
        ],
        out_shape=[
            jax.ShapeDtypeStruct((T * N, OUT), jnp.float32),
            jax.ShapeDtypeStruct((T * N, 2), jnp.float32),
        ],
    )(x2d, weight, attm)


def _edge_core_jnp(src, dst, valid, xw, s_dst, s_src):
    deg = jax.ops.segment_sum(valid, dst, num_segments=N)
    alpha = s_dst[dst] + s_src[src]
    alpha = jnp.maximum(alpha, NEG * alpha)
    p = jnp.clip(1.0 / (1.0 + jnp.exp(-alpha)), 0.01, 0.99)
    a = p * valid / (deg[dst] + 1e-16)
    out = jax.ops.segment_sum(a[:, None] * xw[src], dst, num_segments=N)
    return out, p


RPT = NROW // 16
RPW = NROW // NW
NSL = NPAD // 16
ROWS_T = N // 16


def _att_body(enc_hbm, sd_hbm, ss_hbm, p_hbm, a_hbm,
              sd_v, ss_v, inv_v, deg_v, comb_v, invsl_v, encb, pb, ab,
              spm_deg, spm_inv):
    cid = lax.axis_index("c")
    sid = lax.axis_index("s")
    wid = sid * 2 + cid
    zed = jnp.zeros((16,), jnp.float32)

    pltpu.sync_copy(sd_hbm, sd_v)
    pltpu.sync_copy(ss_hbm, ss_v)

    def zero_deg(i, _):
        deg_v[pl.ds(i * 16, 16)] = zed
        return 0
    lax.fori_loop(0, NPAD // 16, zero_deg, 0)

    def dega(ch, _):
        row0 = sid * RPT + ch * 2
        pltpu.sync_copy(enc_hbm.at[pl.ds(row0, 2)], encb)
        for j in range(2):
            for k in range(8):
                enc = encb[j, pl.ds(k * 16, 16)]
                d16 = lax.shift_right_logical(enc, 1) & 16383
                v16 = (enc & 1).astype(jnp.float32)
                plsc.addupdate_scatter(deg_v, [d16], v16)
        return 0
    lax.fori_loop(0, RPT // 2, dega, 0)

    pltpu.sync_copy(deg_v, spm_deg.at[sid])
    plsc.subcore_barrier()

    pltpu.sync_copy(spm_deg.at[:, pl.ds(sid * NSL, NSL)], comb_v)

    def comb(jj, _):
        acc = comb_v[0, pl.ds(jj * 16, 16)]
        for i in range(1, 16):
            acc = acc + comb_v[i, pl.ds(jj * 16, 16)]
        invsl_v[pl.ds(jj * 16, 16)] = 1.0 / (acc + 1e-16)
        return 0
    lax.fori_loop(0, NSL // 16, comb, 0)
    pltpu.sync_copy(invsl_v, spm_inv.at[pl.ds(sid * NSL, NSL)])
    plsc.subcore_barrier()
    pltpu.sync_copy(spm_inv, inv_v)

    def attn(ch, _):
        row0 = wid * RPW + ch * 2
        pltpu.sync_copy(enc_hbm.at[pl.ds(row0, 2)], encb)
        for j in range(2):
            for k in range(8):
                enc = encb[j, pl.ds(k * 16, 16)]
                s16 = lax.shift_right_logical(enc, 15)
                d16 = lax.shift_right_logical(enc, 1) & 16383
                v16 = (enc & 1).astype(jnp.float32)
                alpha = plsc.load_gather(sd_v, [d16]) + plsc.load_gather(ss_v, [s16])
                alpha = jnp.maximum(alpha, NEG * alpha)
                pv = 1.0 / (1.0 + jnp.exp(-alpha))
                pv = jnp.minimum(jnp.maximum(pv, 0.01), 0.99)
                pb[j, pl.ds(k * 16, 16)] = pv
                ab[j, pl.ds(k * 16, 16)] = pv * v16 * plsc.load_gather(inv_v, [d16])
        pltpu.sync_copy(pb, p_hbm.at[pl.ds(row0, 2)])
        pltpu.sync_copy(ab, a_hbm.at[pl.ds(row0, 2)])
        return 0
    lax.fori_loop(0, RPW // 2, attn, 0)


def _msg_body(enc_hbm, xw_hbm, a_hbm, parts_hbm,
              encb, srcb, dstb, af0, rows0,
              spm_out, gsem0):
    cid = lax.axis_index("c")
    sid = lax.axis_index("s")
    wid = sid * 2 + cid
    zed = jnp.zeros((16,), jnp.float32)

    def zrows(i, _):
        for k in range(8):
            rows0[i, pl.ds(k * 16, 16)] = zed
        return 0
    lax.fori_loop(0, CHUNK, zrows, 0)
    pltpu.sync_copy(rows0, spm_out.at[pl.ds(sid * ROWS_T, CHUNK)])
    pltpu.sync_copy(rows0, spm_out.at[pl.ds(sid * ROWS_T + CHUNK, CHUNK)])
    pltpu.sync_copy(rows0.at[pl.ds(0, ROWS_T - 2 * CHUNK)],
                    spm_out.at[pl.ds(sid * ROWS_T + 2 * CHUNK, ROWS_T - 2 * CHUNK)])
    plsc.subcore_barrier()

    def chunk_body(ch, _):
        row0 = wid * RPW + ch * 2
        pltpu.sync_copy(enc_hbm.at[pl.ds(row0, 2)], encb)
        for j in range(2):
            for k in range(8):
                enc = encb[j, pl.ds(k * 16, 16)]
                srcb[j, pl.ds(k * 16, 16)] = lax.shift_right_logical(enc, 15)
                dstb[j, pl.ds(k * 16, 16)] = lax.shift_right_logical(enc, 1) & 16383
        ds_g = [pltpu.async_copy(xw_hbm.at[srcb.at[j]],
                                 rows0.at[pl.ds(j * 128, 128)], gsem0)
                for j in range(2)]
        for j in range(2):
            pltpu.sync_copy(a_hbm.at[row0 + j], af0.at[pl.ds(j * 128, 128)])
        for d in ds_g:
            d.wait()

        def scale(e4, _):
            for u in range(4):
                e = e4 * 4 + u
                spl = plsc.load_gather(af0, [jnp.full((16,), 0, jnp.int32) + e])
                for k in range(8):
                    rows0[e, pl.ds(k * 16, 16)] = rows0[e, pl.ds(k * 16, 16)] * spl
            return 0
        lax.fori_loop(0, CHUNK // 4, scale, 0)
        for j in range(2):
            pltpu.sync_copy(rows0.at[pl.ds(j * 128, 128)],
                            spm_out.at[dstb.at[j]], add=True)
        return 0
    lax.fori_loop(0, RPW // 2, chunk_body, 0)
    plsc.subcore_barrier()

    for off, sz in ((0, CHUNK), (CHUNK, CHUNK), (2 * CHUNK, ROWS_T - 2 * CHUNK)):
        pltpu.sync_copy(spm_out.at[pl.ds(sid * ROWS_T + off, sz)],
                        rows0.at[pl.ds(0, sz)])
        pltpu.sync_copy(rows0.at[pl.ds(0, sz)],
                        parts_hbm.at[cid, pl.ds(sid * ROWS_T + off, sz)])


_SC_PARAMS = pltpu.CompilerParams(use_tc_tiling_on_sc=False,
                                  needs_layout_passes=False)


def _edge_core_sc(enc2d, xw, sd_pad, ss_pad):
    mesh = plsc.VectorSubcoreMesh(core_axis_name="c", subcore_axis_name="s")
    p2d, a2d = pl.kernel(
        _att_body,
        out_type=[
            jax.ShapeDtypeStruct((NROW, 128), jnp.float32),
            jax.ShapeDtypeStruct((NROW, 128), jnp.float32),
        ],
        mesh=mesh,
        scratch_types=[
            pltpu.VMEM((NPAD,), jnp.float32),
            pltpu.VMEM((NPAD,), jnp.float32),
            pltpu.VMEM((NPAD,), jnp.float32),
            pltpu.VMEM((NPAD,), jnp.float32),
            pltpu.VMEM((16, NSL), jnp.float32),
            pltpu.VMEM((NSL,), jnp.float32),
            pltpu.VMEM((2, 128), jnp.int32),
            pltpu.VMEM((2, 128), jnp.float32),
            pltpu.VMEM((2, 128), jnp.float32),
            pltpu.VMEM_SHARED((16, NPAD), jnp.float32),
            pltpu.VMEM_SHARED((NPAD,), jnp.float32),
        ],
        compiler_params=_SC_PARAMS,
    )(enc2d, sd_pad, ss_pad)
    parts = pl.kernel(
        _msg_body,
        out_type=jax.ShapeDtypeStruct((2, N, 128), jnp.float32),
        mesh=mesh,
        scratch_types=[
            pltpu.VMEM((2, 128), jnp.int32),
            pltpu.VMEM((2, 128), jnp.int32),
            pltpu.VMEM((2, 128), jnp.int32),
            pltpu.VMEM((CHUNK,), jnp.float32),
            pltpu.VMEM((CHUNK, 128), jnp.float32),
            pltpu.VMEM_SHARED((N, 128), jnp.float32),
            pltpu.SemaphoreType.DMA,
        ],
        compiler_params=_SC_PARAMS,
    )(enc2d, xw, a2d)
    return parts, p2d


def _kl_body(p_ref, v_ref, kl_ref, vs_ref):
    j = pl.program_id(1)

    @pl.when(j == 0)
    def _():
        kl_ref[...] = jnp.zeros_like(kl_ref)
        vs_ref[...] = jnp.zeros_like(vs_ref)

    p = p_ref[...]
    v = v_ref[...]
    kl = p * (jnp.log(p) - _LOG_Q) + (1.0 - p) * (jnp.log1p(-p) - _LOG_1MQ)
    kl_ref[...] += jnp.sum(kl * v).reshape(1, 1, 1)
    vs_ref[...] += jnp.sum(v).reshape(1, 1, 1)


def _kl_sums(p3d, v3d):
    BR = 64
    g = NROW // BR
    return pl.pallas_call(
        _kl_body,
        grid=(T, g),
        in_specs=[
            pl.BlockSpec((1, BR, 128), lambda t, j: (t, j, 0)),
            pl.BlockSpec((1, BR, 128), lambda t, j: (t, j, 0)),
        ],
        out_specs=[
            pl.BlockSpec((1, 1, 1), lambda t, j: (t, 0, 0)),
            pl.BlockSpec((1, 1, 1), lambda t, j: (t, 0, 0)),
        ],
        out_shape=[
            jax.ShapeDtypeStruct((T, 1, 1), jnp.float32),
            jax.ShapeDtypeStruct((T, 1, 1), jnp.float32),
        ],
    )(p3d, v3d)


def _mix_body(parts_ref, bias_ref, out_ref):
    x = parts_ref[...]
    b = bias_ref[...]
    raw0 = x[0, 0] + x[0, 1] + b
    raw1 = x[1, 0] + x[1, 1] + b
    raw2 = x[2, 0] + x[2, 1] + b
    o0 = raw0
    o1 = 0.5 * AGG * o0 + (1.0 - AGG) * raw1
    o2 = 0.5 * AGG / 2.0 * o0 + _W1 * AGG / 2.0 * o1 + (1.0 - AGG) * raw2
    out_ref[...] = jnp.stack([o0, o1, o2], axis=0)


def _mix(parts, bias2d):
    B = 400
    g = N // B
    return pl.pallas_call(
        _mix_body,
        grid=(g,),
        in_specs=[
            pl.BlockSpec((T, 2, B, 128), lambda i: (0, 0, i, 0)),
            pl.BlockSpec((1, 128), lambda i: (0, 0)),
        ],
        out_specs=pl.BlockSpec((T, B, 128), lambda i: (0, i, 0)),
        out_shape=jax.ShapeDtypeStruct((T, N, 128), jnp.float32),
    )(parts, bias2d)


def kernel(x_all, edge_index_all, weight, att, bias):
    x2d = x_all.reshape(T * N, D)
    attm = jnp.stack([att[0, 0, :OUT], att[0, 0, OUT:]], axis=1)
    xw2d, s2d = _prep(x2d, weight, attm)
    xw_all = xw2d.reshape(T, N, OUT)
    s_dst_all = s2d[:, 0].reshape(T, N)
    s_src_all = s2d[:, 1].reshape(T, N)

    loop = jnp.arange(N, dtype=jnp.int32)
    pad = jnp.zeros((EPAD - EP,), dtype=jnp.int32)
    parts = []
    ps = []
    vs = []
    encs = []
    valids = []
    for t in range(T):
        e = edge_index_all[t]
        src = jnp.concatenate([e[0], loop, pad])
        dst = jnp.concatenate([e[1], loop, pad])
        valid = jnp.concatenate([
            (e[0] != e[1]).astype(jnp.float32),
            jnp.ones((N,), jnp.float32),
            jnp.zeros((EPAD - EP,), jnp.float32),
        ])
        encs.append((src * 32768 + dst * 2 + valid.astype(jnp.int32)).reshape(NROW, 128))
        valids.append(valid.reshape(NROW, 128))
    for t in range(T):
        sd_pad = jnp.pad(s_dst_all[t], (0, NPAD - N))
        ss_pad = jnp.pad(s_src_all[t], (0, NPAD - N))
        parts_t, p2d = _edge_core_sc(encs[t], xw_all[t], sd_pad, ss_pad)
        parts.append(parts_t)
        ps.append(p2d)

    parts = jnp.stack(parts, axis=0)
    p3d = jnp.stack(ps, axis=0)
    v3d = jnp.stack(valids, axis=0)

    kl_sum, v_sum = _kl_sums(p3d, v3d)
    skl_mean = jnp.mean(kl_sum[:, 0, 0] / v_sum[:, 0, 0])
    out_list = _mix(parts, bias.reshape(1, 128))
    zero = jnp.zeros((), jnp.float32)
    return (out_list, zero, skl_mean, zero)
